# baseline (speedup 1.0000x reference)
import jax
import jax.numpy as jnp
from jax.experimental import pallas as pl
from jax.experimental.pallas import tpu as pltpu

N_ROWS = 16384
N_COLS = 1000
Q_ROWS = 32
P1_BLOCK = 2048
P1_NB = N_ROWS // P1_BLOCK
P2_BLOCK = 2048
P2_NB = N_ROWS // P2_BLOCK


def _colsum_body(ptr_ref, probs_ref, queue_ref, denom_ref):
    i = pl.program_id(0)

    @pl.when(i == 0)
    def _init():
        denom_ref[...] = jnp.zeros_like(denom_ref)

    denom_ref[...] += jnp.sum(probs_ref[...], axis=0, keepdims=True)

    @pl.when(i == P1_NB - 1)
    def _finalize():
        m = denom_ref[...] * (1.0 / N_ROWS)
        ptr = ptr_ref[0]
        row_ids = jax.lax.broadcasted_iota(jnp.int32, (Q_ROWS, N_COLS), 0)
        masked_q = jnp.where(row_ids == ptr, 0.0, queue_ref[...])
        qsum = jnp.sum(masked_q, axis=0, keepdims=True)
        denom_ref[...] = (qsum + m) * (1.0 / Q_ROWS)


def _normalize_body(probs_ref, denom_ref, out_ref):
    t = probs_ref[...] / denom_ref[...]
    s = jnp.sum(t, axis=1, keepdims=True)
    out_ref[...] = t / s


def kernel(probs, DA_queue, DA_ptr):
    ptr = jnp.asarray(DA_ptr, dtype=jnp.int32).reshape((1,))

    denom = pl.pallas_call(
        _colsum_body,
        grid=(P1_NB,),
        in_specs=[
            pl.BlockSpec(memory_space=pltpu.SMEM),
            pl.BlockSpec((P1_BLOCK, N_COLS), lambda i: (i, 0)),
            pl.BlockSpec((Q_ROWS, N_COLS), lambda i: (0, 0)),
        ],
        out_specs=pl.BlockSpec((1, N_COLS), lambda i: (0, 0)),
        out_shape=jax.ShapeDtypeStruct((1, N_COLS), jnp.float32),
    )(ptr, probs, DA_queue)

    out = pl.pallas_call(
        _normalize_body,
        grid=(P2_NB,),
        in_specs=[
            pl.BlockSpec((P2_BLOCK, N_COLS), lambda i: (i, 0)),
            pl.BlockSpec((1, N_COLS), lambda i: (0, 0)),
        ],
        out_specs=pl.BlockSpec((P2_BLOCK, N_COLS), lambda i: (i, 0)),
        out_shape=jax.ShapeDtypeStruct((N_ROWS, N_COLS), jnp.float32),
    )(probs, denom)

    return jax.lax.stop_gradient(out)


# speedup vs baseline: 1.0048x; 1.0048x over previous
import jax
import jax.numpy as jnp
from jax.experimental import pallas as pl
from jax.experimental.pallas import tpu as pltpu

N_ROWS = 16384
N_COLS = 1000
Q_ROWS = 32
P1_BLOCK = 2048
P1_NB = N_ROWS // P1_BLOCK
P2_BLOCK = 2048
P2_NB = N_ROWS // P2_BLOCK


def _colsum_body(ptr_ref, probs_ref, queue_ref, denom_ref):
    i = pl.program_id(0)

    @pl.when(i == 0)
    def _init():
        denom_ref[...] = jnp.zeros_like(denom_ref)

    denom_ref[...] += jnp.sum(probs_ref[...], axis=0, keepdims=True)

    @pl.when(i == P1_NB - 1)
    def _finalize():
        m = denom_ref[...] * (1.0 / N_ROWS)
        ptr = ptr_ref[0]
        row_ids = jax.lax.broadcasted_iota(jnp.int32, (Q_ROWS, N_COLS), 0)
        masked_q = jnp.where(row_ids == ptr, 0.0, queue_ref[...])
        qsum = jnp.sum(masked_q, axis=0, keepdims=True)
        denom_ref[...] = (qsum + m) * (1.0 / Q_ROWS)


def _normalize_body(probs_ref, denom_ref, out_hbm, b0, b1, s0, s1):
    i = pl.program_id(0)
    bufs = (b0, b1)
    sems = (s0, s1)

    def copy_for(step, slot):
        return pltpu.make_async_copy(
            bufs[slot],
            out_hbm.at[pl.ds(step * P2_BLOCK, P2_BLOCK), :],
            sems[slot],
        )

    for slot in range(2):

        @pl.when((i >= 2) & (i % 2 == slot))
        def _wait_prev():
            copy_for(i - 2, slot).wait()

        @pl.when(i % 2 == slot)
        def _compute_and_send():
            t = probs_ref[...] / denom_ref[...]
            s = jnp.sum(t, axis=1, keepdims=True)
            bufs[slot][...] = t / s
            copy_for(i, slot).start()

    @pl.when(i == P2_NB - 1)
    def _drain():
        copy_for(P2_NB - 2, (P2_NB - 2) % 2).wait()
        copy_for(P2_NB - 1, (P2_NB - 1) % 2).wait()


def kernel(probs, DA_queue, DA_ptr):
    ptr = jnp.asarray(DA_ptr, dtype=jnp.int32).reshape((1,))

    denom = pl.pallas_call(
        _colsum_body,
        grid=(P1_NB,),
        in_specs=[
            pl.BlockSpec(memory_space=pltpu.SMEM),
            pl.BlockSpec((P1_BLOCK, N_COLS), lambda i: (i, 0)),
            pl.BlockSpec((Q_ROWS, N_COLS), lambda i: (0, 0)),
        ],
        out_specs=pl.BlockSpec((1, N_COLS), lambda i: (0, 0)),
        out_shape=jax.ShapeDtypeStruct((1, N_COLS), jnp.float32),
    )(ptr, probs, DA_queue)

    out = pl.pallas_call(
        _normalize_body,
        grid=(P2_NB,),
        in_specs=[
            pl.BlockSpec((P2_BLOCK, N_COLS), lambda i: (i, 0)),
            pl.BlockSpec((1, N_COLS), lambda i: (0, 0)),
        ],
        out_specs=pl.BlockSpec(memory_space=pl.ANY),
        out_shape=jax.ShapeDtypeStruct((N_ROWS, N_COLS), jnp.float32),
        scratch_shapes=[
            pltpu.VMEM((P2_BLOCK, N_COLS), jnp.float32),
            pltpu.VMEM((P2_BLOCK, N_COLS), jnp.float32),
            pltpu.SemaphoreType.DMA,
            pltpu.SemaphoreType.DMA,
        ],
    )(probs, denom)

    return jax.lax.stop_gradient(out)
